# baseline (speedup 1.0000x reference)
import functools

import jax
import jax.numpy as jnp
from jax import lax
from jax.experimental import pallas as pl
from jax.experimental.pallas import tpu as pltpu
from jax.experimental.pallas import tpu_sc as plsc

N = 5000
NCLS = 21
NOUT = NCLS - 1
LANES = 16
NPAD = 5072
NV = NPAD // LANES
UNROLL = 6
IOU_T = 0.5
SCORE_T = 0.5


def _sload(ref, i):
    return ref[pl.ds(i, LANES)][0]


def _nms_body(comp, probs, order, obox, olab, osco,
              cy1, cx1, cy2, cx2, ps, sord,
              sy1, sx1, sy2, sx2, packed, skeep, korig, obx, ols, oss):
    wid = lax.axis_index("s") * 2 + lax.axis_index("c")

    @pl.when(wid < NOUT)
    def _work():
        cls = wid + 1
        lanes = lax.iota(jnp.int32, LANES)

        pltpu.sync_copy(comp.at[cls, 0], cy1)
        pltpu.sync_copy(comp.at[cls, 1], cx1)
        pltpu.sync_copy(comp.at[cls, 2], cy2)
        pltpu.sync_copy(comp.at[cls, 3], cx2)
        pltpu.sync_copy(probs.at[cls], ps)
        pltpu.sync_copy(order.at[wid], sord)

        def init_body(k, m):
            base = k * LANES
            idx = sord[pl.ds(base, LANES)]
            sg = plsc.load_gather(ps, [idx])
            y1 = plsc.load_gather(cy1, [idx])
            x1 = plsc.load_gather(cx1, [idx])
            y2 = plsc.load_gather(cy2, [idx])
            x2 = plsc.load_gather(cx2, [idx])
            sy1[pl.ds(base, LANES)] = y1
            sx1[pl.ds(base, LANES)] = x1
            sy2[pl.ds(base, LANES)] = y2
            sx2[pl.ds(base, LANES)] = x2
            p5 = (lanes + jnp.full((LANES,), base)) * 5
            plsc.store_scatter(packed, [p5], y1)
            plsc.store_scatter(packed, [p5 + 1], x1)
            plsc.store_scatter(packed, [p5 + 2], y2)
            plsc.store_scatter(packed, [p5 + 3], x2)
            plsc.store_scatter(packed, [p5 + 4], (y2 - y1) * (x2 - x1))
            valid = sg > SCORE_T
            skeep[pl.ds(base, LANES)] = jnp.where(valid, 1.0, 0.0)
            return m + jnp.sum(valid.astype(jnp.int32))

        m = lax.fori_loop(0, NV, init_body, jnp.int32(0))
        mv = (m + LANES - 1) // LANES

        def i_body(i, carry):
            ki = _sload(skeep, i)

            @pl.when(ki > 0.0)
            def _suppress():
                pv = packed[pl.ds(i * 5, LANES)]
                y1b = jnp.full((LANES,), pv[0])
                x1b = jnp.full((LANES,), pv[1])
                y2b = jnp.full((LANES,), pv[2])
                x2b = jnp.full((LANES,), pv[3])
                ab = jnp.full((LANES,), pv[4])
                zeros = jnp.zeros((LANES,), jnp.float32)

                def overlap(base):
                    y1j = sy1[pl.ds(base, LANES)]
                    x1j = sx1[pl.ds(base, LANES)]
                    y2j = sy2[pl.ds(base, LANES)]
                    x2j = sx2[pl.ds(base, LANES)]
                    aj = (y2j - y1j) * (x2j - x1j)
                    ih = jnp.maximum(
                        jnp.minimum(y2b, y2j) - jnp.maximum(y1b, y1j), 0.0)
                    iw = jnp.maximum(
                        jnp.minimum(x2b, x2j) - jnp.maximum(x1b, x1j), 0.0)
                    inter = ih * iw
                    return 3.0 * inter > ab + aj

                jb0 = i // LANES
                hbase = jb0 * LANES
                hidx = lanes + jnp.full((LANES,), hbase)
                hsup = overlap(hbase) & (hidx > jnp.full((LANES,), i))
                plsc.store_scatter(skeep, [hidx], zeros, mask=hsup)

                @plsc.parallel_loop(jb0 + 1, mv, step=1, unroll=UNROLL)
                def _jloop(jb):
                    base = jb * LANES
                    sup = overlap(base)
                    plsc.store_scatter(
                        skeep, [lanes + jnp.full((LANES,), base)],
                        zeros, mask=sup)

            return carry

        lax.fori_loop(0, m, i_body, jnp.int32(0))

        def scat_body(k, c):
            base = k * LANES
            idx = sord[pl.ds(base, LANES)]
            plsc.store_scatter(korig, [idx], skeep[pl.ds(base, LANES)])
            return c

        lax.fori_loop(0, NV, scat_body, jnp.int32(0))

        lblv = (cls - 1).astype(jnp.float32)

        def out_body(k, c):
            base = k * LANES
            kf = korig[pl.ds(base, LANES)]
            flat = (lanes + jnp.full((LANES,), base)) * 4
            plsc.store_scatter(obx, [flat], cy1[pl.ds(base, LANES)] * kf)
            plsc.store_scatter(obx, [flat + 1], cx1[pl.ds(base, LANES)] * kf)
            plsc.store_scatter(obx, [flat + 2], cy2[pl.ds(base, LANES)] * kf)
            plsc.store_scatter(obx, [flat + 3], cx2[pl.ds(base, LANES)] * kf)
            ols[pl.ds(base, LANES)] = kf * jnp.full((LANES,), lblv)
            oss[pl.ds(base, LANES)] = kf * ps[pl.ds(base, LANES)]
            return c

        lax.fori_loop(0, NV, out_body, jnp.int32(0))

        pltpu.sync_copy(obx.at[pl.ds(0, N * 4)],
                        obox.at[pl.ds(wid * N * 4, N * 4)])
        pltpu.sync_copy(ols.at[pl.ds(0, N)], olab.at[pl.ds(wid * N, N)])
        pltpu.sync_copy(oss.at[pl.ds(0, N)], osco.at[pl.ds(wid * N, N)])


@jax.jit
def _nms_call(comp, probs, order):
    mesh = plsc.VectorSubcoreMesh(
        core_axis_name="c", subcore_axis_name="s",
        num_cores=2, num_subcores=16)
    f32 = jnp.float32
    kern = pl.kernel(
        _nms_body,
        out_type=(
            jax.ShapeDtypeStruct((NOUT * N * 4,), f32),
            jax.ShapeDtypeStruct((NOUT * N,), f32),
            jax.ShapeDtypeStruct((NOUT * N,), f32),
        ),
        mesh=mesh,
        scratch_types=[
            pltpu.VMEM((NPAD,), f32),
            pltpu.VMEM((NPAD,), f32),
            pltpu.VMEM((NPAD,), f32),
            pltpu.VMEM((NPAD,), f32),
            pltpu.VMEM((NPAD,), f32),
            pltpu.VMEM((NPAD,), jnp.int32),
            pltpu.VMEM((NPAD,), f32),
            pltpu.VMEM((NPAD,), f32),
            pltpu.VMEM((NPAD,), f32),
            pltpu.VMEM((NPAD,), f32),
            pltpu.VMEM((NPAD * 5,), f32),
            pltpu.VMEM((NPAD,), f32),
            pltpu.VMEM((NPAD,), f32),
            pltpu.VMEM((NPAD * 4,), f32),
            pltpu.VMEM((NPAD,), f32),
            pltpu.VMEM((NPAD,), f32),
        ],
        compiler_params=pltpu.CompilerParams(needs_layout_passes=False),
        name="sc_nms",
    )
    return kern(comp, probs, order)


def kernel(raw_boxes, raw_probs):
    boxes = raw_boxes.reshape(N, NCLS, 4)
    comp = jnp.transpose(boxes, (1, 2, 0))
    comp = jnp.pad(comp, ((0, 0), (0, 0), (0, NPAD - N)))
    probs = jnp.pad(raw_probs, ((0, 0), (0, NPAD - N)))
    order = jnp.argsort(-probs[1:], axis=1).astype(jnp.int32)
    bflat, labels, scores = _nms_call(comp, probs, order)
    return bflat.reshape(NOUT * N, 4), labels, scores


# speedup vs baseline: 1.0084x; 1.0084x over previous
import jax
import jax.numpy as jnp
from jax import lax
from jax.experimental import pallas as pl
from jax.experimental.pallas import tpu as pltpu
from jax.experimental.pallas import tpu_sc as plsc

N = 5000
NCLS = 21
NOUT = NCLS - 1
LANES = 16
NPAD = 5072
NV = NPAD // LANES
UNROLL = 4
SCORE_T = 0.5


def _sload(ref, i):
    return ref[pl.ds(i, LANES)][0]


def _nms_body(comp, probs, order, obox, olab, osco,
              cy1, cx1, cy2, cx2, ps, sord,
              sy1, sx1, sy2, sx2, packed, skeep, korig, obx, ols, oss):
    wid = lax.axis_index("s") * 2 + lax.axis_index("c")

    @pl.when(wid < NOUT)
    def _work():
        cls = wid + 1
        lanes = lax.iota(jnp.int32, LANES)

        pltpu.sync_copy(comp.at[cls, 0], cy1)
        pltpu.sync_copy(comp.at[cls, 1], cx1)
        pltpu.sync_copy(comp.at[cls, 2], cy2)
        pltpu.sync_copy(comp.at[cls, 3], cx2)
        pltpu.sync_copy(probs.at[cls], ps)
        pltpu.sync_copy(order.at[wid], sord)

        def init_body(k, m):
            base = k * LANES
            idx = sord[pl.ds(base, LANES)]
            sg = plsc.load_gather(ps, [idx])
            y1 = plsc.load_gather(cy1, [idx])
            x1 = plsc.load_gather(cx1, [idx])
            y2 = plsc.load_gather(cy2, [idx])
            x2 = plsc.load_gather(cx2, [idx])
            sy1[pl.ds(base, LANES)] = y1
            sx1[pl.ds(base, LANES)] = x1
            sy2[pl.ds(base, LANES)] = y2
            sx2[pl.ds(base, LANES)] = x2
            p5 = (lanes + jnp.full((LANES,), base)) * 5
            plsc.store_scatter(packed, [p5], y1)
            plsc.store_scatter(packed, [p5 + 1], x1)
            plsc.store_scatter(packed, [p5 + 2], y2)
            plsc.store_scatter(packed, [p5 + 3], x2)
            plsc.store_scatter(packed, [p5 + 4], (y2 - y1) * (x2 - x1))
            valid = sg > SCORE_T
            skeep[pl.ds(base, LANES)] = jnp.where(valid, 1.0, 0.0)
            return m + jnp.sum(valid.astype(jnp.int32))

        m = lax.fori_loop(0, NV, init_body, jnp.int32(0))
        mv = (m + LANES - 1) // LANES

        def i_body(i, carry):
            ki = _sload(skeep, i)

            @pl.when(ki > 0.0)
            def _suppress():
                pv = packed[pl.ds(i * 5, LANES)]
                y1b = jnp.full((LANES,), pv[0])
                x1b = jnp.full((LANES,), pv[1])
                y2b = jnp.full((LANES,), pv[2])
                x2b = jnp.full((LANES,), pv[3])
                ab = jnp.full((LANES,), pv[4])
                zeros = jnp.zeros((LANES,), jnp.float32)

                def overlap(base):
                    y1j = sy1[pl.ds(base, LANES)]
                    x1j = sx1[pl.ds(base, LANES)]
                    y2j = sy2[pl.ds(base, LANES)]
                    x2j = sx2[pl.ds(base, LANES)]
                    aj = (y2j - y1j) * (x2j - x1j)
                    ih = jnp.maximum(
                        jnp.minimum(y2b, y2j) - jnp.maximum(y1b, y1j), 0.0)
                    iw = jnp.maximum(
                        jnp.minimum(x2b, x2j) - jnp.maximum(x1b, x1j), 0.0)
                    inter = ih * iw
                    return 3.0 * inter > ab + aj

                jb0 = i // LANES
                hbase = jb0 * LANES
                hidx = lanes + jnp.full((LANES,), hbase)
                hsup = overlap(hbase) & (hidx > jnp.full((LANES,), i))
                plsc.store_scatter(skeep, [hidx], zeros, mask=hsup)

                @plsc.parallel_loop(jb0 + 1, mv, step=1, unroll=UNROLL)
                def _jloop(jb):
                    base = jb * LANES
                    sup = overlap(base)
                    plsc.store_scatter(
                        skeep, [lanes + jnp.full((LANES,), base)],
                        zeros, mask=sup)

            return carry

        lax.fori_loop(0, m, i_body, jnp.int32(0))

        def scat_body(k, c):
            base = k * LANES
            idx = sord[pl.ds(base, LANES)]
            plsc.store_scatter(korig, [idx], skeep[pl.ds(base, LANES)])
            return c

        lax.fori_loop(0, NV, scat_body, jnp.int32(0))

        lblv = (cls - 1).astype(jnp.float32)

        def out_body(k, c):
            base = k * LANES
            kf = korig[pl.ds(base, LANES)]
            flat = (lanes + jnp.full((LANES,), base)) * 4
            plsc.store_scatter(obx, [flat], cy1[pl.ds(base, LANES)] * kf)
            plsc.store_scatter(obx, [flat + 1], cx1[pl.ds(base, LANES)] * kf)
            plsc.store_scatter(obx, [flat + 2], cy2[pl.ds(base, LANES)] * kf)
            plsc.store_scatter(obx, [flat + 3], cx2[pl.ds(base, LANES)] * kf)
            ols[pl.ds(base, LANES)] = kf * jnp.full((LANES,), lblv)
            oss[pl.ds(base, LANES)] = kf * ps[pl.ds(base, LANES)]
            return c

        lax.fori_loop(0, NV, out_body, jnp.int32(0))

        pltpu.sync_copy(obx.at[pl.ds(0, N * 4)],
                        obox.at[pl.ds(wid * N * 4, N * 4)])
        pltpu.sync_copy(ols.at[pl.ds(0, N)], olab.at[pl.ds(wid * N, N)])
        pltpu.sync_copy(oss.at[pl.ds(0, N)], osco.at[pl.ds(wid * N, N)])


@jax.jit
def _nms_call(comp, probs, order):
    mesh = plsc.VectorSubcoreMesh(
        core_axis_name="c", subcore_axis_name="s",
        num_cores=2, num_subcores=16)
    f32 = jnp.float32
    kern = pl.kernel(
        _nms_body,
        out_type=(
            jax.ShapeDtypeStruct((NOUT * N * 4,), f32),
            jax.ShapeDtypeStruct((NOUT * N,), f32),
            jax.ShapeDtypeStruct((NOUT * N,), f32),
        ),
        mesh=mesh,
        scratch_types=[
            pltpu.VMEM((NPAD,), f32),
            pltpu.VMEM((NPAD,), f32),
            pltpu.VMEM((NPAD,), f32),
            pltpu.VMEM((NPAD,), f32),
            pltpu.VMEM((NPAD,), f32),
            pltpu.VMEM((NPAD,), jnp.int32),
            pltpu.VMEM((NPAD,), f32),
            pltpu.VMEM((NPAD,), f32),
            pltpu.VMEM((NPAD,), f32),
            pltpu.VMEM((NPAD,), f32),
            pltpu.VMEM((NPAD * 5,), f32),
            pltpu.VMEM((NPAD,), f32),
            pltpu.VMEM((NPAD,), f32),
            pltpu.VMEM((NPAD * 4,), f32),
            pltpu.VMEM((NPAD,), f32),
            pltpu.VMEM((NPAD,), f32),
        ],
        compiler_params=pltpu.CompilerParams(needs_layout_passes=False),
        name="sc_nms",
    )
    return kern(comp, probs, order)


def kernel(raw_boxes, raw_probs):
    boxes = raw_boxes.reshape(N, NCLS, 4)
    comp = jnp.transpose(boxes, (1, 2, 0))
    comp = jnp.pad(comp, ((0, 0), (0, 0), (0, NPAD - N)))
    probs = jnp.pad(raw_probs, ((0, 0), (0, NPAD - N)))
    order = jnp.argsort(-probs[1:], axis=1).astype(jnp.int32)
    bflat, labels, scores = _nms_call(comp, probs, order)
    return bflat.reshape(NOUT * N, 4), labels, scores
